# Initial kernel scaffold; baseline (speedup 1.0000x reference)
import functools

import jax
import jax.numpy as jnp
from jax import lax
from jax.experimental import pallas as pl
from jax.experimental.pallas import tpu as pltpu
from jax.experimental.pallas import tpu_sc as plsc

N_NODES = 50000
D_IN = 128
H = 64
E_EDGES = 400000

NC = 2
NS = 16
NPAD = 51200
E_PAD = 400128
K = 128
CW = 80
ET = E_PAD // NS
NBLK = 3
BLK = ET // NBLK
VB = BLK // 16
CAP = ((ET + K - 1) // K + 1) * K


def _agg_body(with_count, W, NQ, Rq, Rt, *refs):
    if with_count:
        (src_hbm, dst_hbm, feat_hbm, zeros_hbm, zeros8_hbm, ones8_hbm,
         out_hbm, cnt_hbm,
         acc, cacc, src_v, dst_v, ssrc, sdst, idxg, idxs, gbuf, zbuf, wbuf,
         z8, w8, ones_v) = refs
    else:
        (src_hbm, dst_hbm, feat_hbm, zeros_hbm,
         out_hbm,
         acc, src_v, dst_v, ssrc, sdst, idxg, idxs, gbuf, zbuf, wbuf) = refs

    cid = lax.axis_index("c")
    sid = lax.axis_index("s")

    pltpu.sync_copy(zeros_hbm, zbuf)
    if with_count:
        pltpu.sync_copy(zeros8_hbm, z8)
        pltpu.sync_copy(ones8_hbm, ones_v)

    tbase = sid * ET
    row0 = sid * Rt
    dummy = jnp.full((16,), Rq, jnp.int32)
    zero16 = jnp.zeros((16,), jnp.int32)

    for q in range(NQ):
        lo = (cid * NQ + q) * Rq
        hi = lo + Rq

        for i in range(Rt // CW):
            pltpu.sync_copy(zbuf, acc.at[pl.ds(row0 + i * CW, CW)])
        if with_count:
            for i in range(Rt // CW):
                pltpu.sync_copy(z8, cacc.at[pl.ds(row0 + i * CW, CW)])
        plsc.subcore_barrier()

        n_sel = jnp.int32(0)
        for blk in range(NBLK):
            base = tbase + blk * BLK
            pltpu.sync_copy(src_hbm.at[pl.ds(base, BLK)], src_v)
            pltpu.sync_copy(dst_hbm.at[pl.ds(base, BLK)], dst_v)

            def fbody(i, n):
                d = dst_v[pl.ds(i * 16, 16)]
                s = src_v[pl.ds(i * 16, 16)]
                m = (d >= lo) & (d < hi)
                plsc.store_compressed(ssrc.at[pl.ds(n, 16)], s, mask=m)
                plsc.store_compressed(sdst.at[pl.ds(n, 16)], d - lo, mask=m)
                return n + jnp.sum(m.astype(jnp.int32))

            n_sel = lax.fori_loop(0, VB, fbody, n_sel)

        for t in range(8):
            ssrc[pl.ds(n_sel + t * 16, 16)] = zero16
            sdst[pl.ds(n_sel + t * 16, 16)] = dummy

        nb = (n_sel + (K - 1)) // K

        def cbody(b, carry):
            pltpu.sync_copy(ssrc.at[pl.ds(b * K, K)], idxg)
            pltpu.sync_copy(sdst.at[pl.ds(b * K, K)], idxs)
            pltpu.sync_copy(feat_hbm.at[idxg], gbuf)
            pltpu.sync_copy(gbuf, acc.at[idxs], add=True)
            if with_count:
                pltpu.sync_copy(ones_v, cacc.at[idxs], add=True)
            return carry

        lax.fori_loop(0, nb, cbody, jnp.int32(0))
        plsc.subcore_barrier()

        gbase = lo + sid * Rt
        for i in range(Rt // CW):
            pltpu.sync_copy(acc.at[pl.ds(row0 + i * CW, CW)], wbuf)
            pltpu.sync_copy(wbuf, out_hbm.at[pl.ds(gbase + i * CW, CW)])
        if with_count:
            for i in range(Rt // CW):
                pltpu.sync_copy(cacc.at[pl.ds(row0 + i * CW, CW)], w8)
                pltpu.sync_copy(w8, cnt_hbm.at[pl.ds(gbase + i * CW, CW)])


def _make_agg(W, NQ, with_count):
    Rq = NPAD // (NC * NQ)
    Rt = Rq // NS
    mesh = plsc.VectorSubcoreMesh(core_axis_name="c", subcore_axis_name="s")

    out_type = [jax.ShapeDtypeStruct((NPAD, W), jnp.float32)]
    if with_count:
        out_type.append(jax.ShapeDtypeStruct((NPAD, 8), jnp.float32))

    scratch = [
        pltpu.VMEM_SHARED((Rq + 16, W), jnp.float32),
    ]
    if with_count:
        scratch.append(pltpu.VMEM_SHARED((Rq + 16, 8), jnp.float32))
    scratch += [
        pltpu.VMEM((BLK,), jnp.int32),
        pltpu.VMEM((BLK,), jnp.int32),
        pltpu.VMEM((CAP,), jnp.int32),
        pltpu.VMEM((CAP,), jnp.int32),
        pltpu.VMEM((K,), jnp.int32),
        pltpu.VMEM((K,), jnp.int32),
        pltpu.VMEM((K, W), jnp.float32),
        pltpu.VMEM((CW, W), jnp.float32),
        pltpu.VMEM((CW, W), jnp.float32),
    ]
    if with_count:
        scratch += [
            pltpu.VMEM((CW, 8), jnp.float32),
            pltpu.VMEM((CW, 8), jnp.float32),
            pltpu.VMEM((K, 8), jnp.float32),
        ]

    return pl.kernel(
        functools.partial(_agg_body, with_count, W, NQ, Rq, Rt),
        out_type=tuple(out_type) if len(out_type) > 1 else out_type[0],
        mesh=mesh,
        scratch_types=scratch,
        name=f"sgcn_agg_w{W}_q{NQ}{'_cnt' if with_count else ''}",
    )


def _mm1_body(x_ref, w_ref, yp_ref, yn_ref, ys_ref):
    y = jnp.dot(x_ref[...], w_ref[...], preferred_element_type=jnp.float32)
    yp_ref[...] = y[:, :H]
    yn_ref[...] = y[:, H:2 * H]
    ys_ref[...] = y[:, 2 * H:]


def _z_body(sp_ref, sn_ref, ys_ref, cp_ref, cn_ref, bb_ref, bu_ref, z_ref):
    cp = jnp.maximum(cp_ref[...][:, 0:1], 1.0)
    cn = jnp.maximum(cn_ref[...][:, 0:1], 1.0)
    zb = sp_ref[...] / cp + ys_ref[...][:, :H] + bb_ref[...]
    zu = sn_ref[...] / cn + ys_ref[...][:, H:] + bu_ref[...]
    z_ref[...] = jnp.tanh(jnp.concatenate([zb, zu], axis=1))


def _f_body(sp_ref, sn_ref, z_ref, cp_ref, cn_ref, wa_ref, wb_ref, wc_ref,
            b2_ref, out_ref):
    cp = jnp.maximum(cp_ref[...][:, 0:1], 1.0)
    cn = jnp.maximum(cn_ref[...][:, 0:1], 1.0)
    a = sp_ref[...] / cp
    b = sn_ref[...] / cn
    acc = jnp.dot(a, wa_ref[...], preferred_element_type=jnp.float32)
    acc += jnp.dot(b, wb_ref[...], preferred_element_type=jnp.float32)
    acc += jnp.dot(z_ref[...], wc_ref[...], preferred_element_type=jnp.float32)
    out_ref[...] = jnp.tanh(acc + b2_ref[...])


_MB = 1000
_GRID = (N_NODES // _MB,)


def _rows(bw):
    return pl.BlockSpec((_MB, bw), lambda i: (i, 0))


def _full(shape):
    return pl.BlockSpec(shape, lambda i: (0, 0))


def kernel(x, pos_edge_index, neg_edge_index, Wb1, bb1, Wu1, bu1,
           Wb2, bb2, Wu2, bu2):
    f32 = jnp.float32
    W1 = jnp.concatenate(
        [Wb1[:D_IN], Wu1[:D_IN], Wb1[D_IN:], Wu1[D_IN:]], axis=1)
    z128 = jnp.zeros((2 * H, 2 * H), f32)
    W2a = z128.at[0:H, 0:H].set(Wb2[0:H]).at[H:2 * H, H:2 * H].set(Wu2[0:H])
    W2b = (z128.at[0:H, H:2 * H].set(Wu2[H:2 * H])
               .at[H:2 * H, 0:H].set(Wb2[H:2 * H]))
    W2c = (z128.at[0:H, 0:H].set(Wb2[2 * H:3 * H])
               .at[H:2 * H, H:2 * H].set(Wu2[2 * H:3 * H]))
    b2 = jnp.concatenate([bb2, bu2]).reshape(1, 2 * H)
    bb1r = bb1.reshape(1, H)
    bu1r = bu1.reshape(1, H)

    padn = E_PAD - E_EDGES
    psrc = jnp.concatenate([pos_edge_index[0], jnp.zeros((padn,), jnp.int32)])
    pdst = jnp.concatenate([pos_edge_index[1],
                            jnp.full((padn,), NPAD, jnp.int32)])
    nsrc = jnp.concatenate([neg_edge_index[0], jnp.zeros((padn,), jnp.int32)])
    ndst = jnp.concatenate([neg_edge_index[1],
                            jnp.full((padn,), NPAD, jnp.int32)])

    zc64 = jnp.zeros((CW, H), f32)
    zc128 = jnp.zeros((CW, 2 * H), f32)
    zc8 = jnp.zeros((CW, 8), f32)
    ones8 = jnp.ones((K, 8), f32)

    yp, yn, ys = pl.pallas_call(
        _mm1_body,
        grid=_GRID,
        in_specs=[_rows(D_IN), _full((D_IN, 4 * H))],
        out_specs=[_rows(H), _rows(H), _rows(2 * H)],
        out_shape=[jax.ShapeDtypeStruct((N_NODES, H), f32),
                   jax.ShapeDtypeStruct((N_NODES, H), f32),
                   jax.ShapeDtypeStruct((N_NODES, 2 * H), f32)],
    )(x, W1)

    agg1 = _make_agg(H, 1, True)
    sp1, cntp = agg1(psrc, pdst, yp, zc64, zc8, ones8)
    sn1, cntn = agg1(nsrc, ndst, yn, zc64, zc8, ones8)

    z = pl.pallas_call(
        _z_body,
        grid=_GRID,
        in_specs=[_rows(H), _rows(H), _rows(2 * H), _rows(8), _rows(8),
                  _full((1, H)), _full((1, H))],
        out_specs=_rows(2 * H),
        out_shape=jax.ShapeDtypeStruct((N_NODES, 2 * H), f32),
    )(sp1, sn1, ys, cntp, cntn, bb1r, bu1r)

    agg2 = _make_agg(2 * H, 2, False)
    sp2 = agg2(psrc, pdst, z, zc128)
    sn2 = agg2(nsrc, ndst, z, zc128)

    out = pl.pallas_call(
        _f_body,
        grid=_GRID,
        in_specs=[_rows(2 * H), _rows(2 * H), _rows(2 * H), _rows(8),
                  _rows(8), _full((2 * H, 2 * H)), _full((2 * H, 2 * H)),
                  _full((2 * H, 2 * H)), _full((1, 2 * H))],
        out_specs=_rows(2 * H),
        out_shape=jax.ShapeDtypeStruct((N_NODES, 2 * H), f32),
    )(sp2, sn2, z, cntp, cntn, W2a, W2b, W2c, b2)
    return out


# speedup vs baseline: 1.0809x; 1.0809x over previous
import functools

import jax
import jax.numpy as jnp
from jax import lax
from jax.experimental import pallas as pl
from jax.experimental.pallas import tpu as pltpu
from jax.experimental.pallas import tpu_sc as plsc

N_NODES = 50000
D = 128
H = 64
E_EDGES = 400000

NC = 2
NS = 16
NQ = 2
NPAD = 50176
E_PAD = 401408
K = 32
CW = 56
CNTW = 16
RQ = NPAD // (NC * NQ)
RT = RQ // NS
ET = E_PAD // NS
NBLK = 49
BLK = ET // NBLK
VB = BLK // 16
SELR = BLK // K + 2
KSH = K.bit_length() - 1


def _agg_body(with_count, *refs):
    if with_count:
        (src_hbm, dst_hbm, feat_hbm, zeros_hbm, zeros8_hbm, ones8_hbm,
         out_hbm, cnt_hbm,
         acc, cacc, src_v, dst_v, ssrc, sdst, gbuf, ones_v) = refs
    else:
        (src_hbm, dst_hbm, feat_hbm, zeros_hbm,
         out_hbm,
         acc, src_v, dst_v, ssrc, sdst, gbuf) = refs

    cid = lax.axis_index("c")
    sid = lax.axis_index("s")

    if with_count:
        pltpu.sync_copy(ones8_hbm, ones_v)

    tbase = sid * ET
    row0 = sid * RT
    dummy = jnp.full((16,), RQ, jnp.int32)
    zero16 = jnp.zeros((16,), jnp.int32)
    lane = lax.iota(jnp.int32, 16)

    for q in range(NQ):
        lo = (cid * NQ + q) * RQ
        hi = lo + RQ

        def zbody(i, c):
            pltpu.sync_copy(zeros_hbm, acc.at[pl.ds(row0 + i * CW, CW)])
            if with_count:
                pltpu.sync_copy(zeros8_hbm,
                                cacc.at[pl.ds(row0 + i * CW, CW)])
            return c

        lax.fori_loop(0, RT // CW, zbody, jnp.int32(0))
        plsc.subcore_barrier()

        def bbody(blk, carry):
            base = tbase + blk * BLK
            pltpu.sync_copy(src_hbm.at[pl.ds(base, BLK)], src_v)
            pltpu.sync_copy(dst_hbm.at[pl.ds(base, BLK)], dst_v)

            def fbody(i, n):
                d = dst_v[pl.ds(i * 16, 16)]
                s = src_v[pl.ds(i * 16, 16)]
                m = (d >= lo) & (d < hi)
                mi = m.astype(jnp.int32)
                offs = plsc.cumsum(mi) + (n - 1)
                orow = lax.shift_right_logical(offs, KSH)
                ocol = lax.bitwise_and(offs, K - 1)
                plsc.store_scatter(ssrc, [orow, ocol], s, mask=m)
                plsc.store_scatter(sdst, [orow, ocol], d - lo, mask=m)
                return n + jnp.sum(mi)

            n_sel = lax.fori_loop(0, VB, fbody, jnp.int32(0))

            for t in range(K // 16):
                offs = lane + (n_sel + t * 16)
                orow = lax.shift_right_logical(offs, KSH)
                ocol = lax.bitwise_and(offs, K - 1)
                plsc.store_scatter(ssrc, [orow, ocol], zero16)
                plsc.store_scatter(sdst, [orow, ocol], dummy)

            nb = (n_sel + (K - 1)) // K

            def cbody(b, carry):
                pltpu.sync_copy(feat_hbm.at[ssrc.at[b]], gbuf)
                pltpu.sync_copy(gbuf, acc.at[sdst.at[b]], add=True)
                if with_count:
                    pltpu.sync_copy(ones_v, cacc.at[sdst.at[b]], add=True)
                return carry

            lax.fori_loop(0, nb, cbody, jnp.int32(0))
            return carry

        lax.fori_loop(0, NBLK, bbody, jnp.int32(0))

        plsc.subcore_barrier()

        gbase = lo + sid * RT

        def wbody(i, c):
            pltpu.sync_copy(acc.at[pl.ds(row0 + i * CW, CW)],
                            out_hbm.at[pl.ds(gbase + i * CW, CW)])
            if with_count:
                pltpu.sync_copy(cacc.at[pl.ds(row0 + i * CW, CW)],
                                cnt_hbm.at[pl.ds(gbase + i * CW, CW)])
            return c

        lax.fori_loop(0, RT // CW, wbody, jnp.int32(0))
        if q + 1 < NQ:
            plsc.subcore_barrier()


def _make_agg(with_count):
    mesh = plsc.VectorSubcoreMesh(core_axis_name="c", subcore_axis_name="s")

    out_type = [jax.ShapeDtypeStruct((NPAD, D), jnp.float32)]
    if with_count:
        out_type.append(jax.ShapeDtypeStruct((NPAD, CNTW), jnp.float32))

    scratch = [
        pltpu.VMEM_SHARED((RQ + 16, D), jnp.float32),
    ]
    if with_count:
        scratch.append(pltpu.VMEM_SHARED((RQ + 16, CNTW), jnp.float32))
    scratch += [
        pltpu.VMEM((BLK,), jnp.int32),
        pltpu.VMEM((BLK,), jnp.int32),
        pltpu.VMEM((SELR, K), jnp.int32),
        pltpu.VMEM((SELR, K), jnp.int32),
        pltpu.VMEM((K, D), jnp.float32),
    ]
    if with_count:
        scratch.append(pltpu.VMEM((K, CNTW), jnp.float32))

    return pl.kernel(
        functools.partial(_agg_body, with_count),
        out_type=tuple(out_type) if len(out_type) > 1 else out_type[0],
        mesh=mesh,
        scratch_types=scratch,
        compiler_params=pltpu.CompilerParams(needs_layout_passes=False),
        name=f"sgcn_agg{'_cnt' if with_count else ''}",
    )


def _mm1_body(x_ref, w_ref, ys_ref):
    ys_ref[...] = jnp.dot(x_ref[...], w_ref[...],
                          preferred_element_type=jnp.float32)


def _z_body(sp_ref, sn_ref, ys_ref, cp_ref, cn_ref, wbt_ref, wut_ref,
            bb_ref, bu_ref, z_ref):
    cp = jnp.maximum(cp_ref[...][:, 0:1], 1.0)
    cn = jnp.maximum(cn_ref[...][:, 0:1], 1.0)
    zb = jnp.dot(sp_ref[...] / cp, wbt_ref[...],
                 preferred_element_type=jnp.float32)
    zu = jnp.dot(sn_ref[...] / cn, wut_ref[...],
                 preferred_element_type=jnp.float32)
    ys = ys_ref[...]
    zb = zb + ys[:, :H] + bb_ref[...]
    zu = zu + ys[:, H:] + bu_ref[...]
    z_ref[...] = jnp.tanh(jnp.concatenate([zb, zu], axis=1))


def _f_body(sp_ref, sn_ref, z_ref, cp_ref, cn_ref, wa_ref, wb_ref, wc_ref,
            b2_ref, out_ref):
    cp = jnp.maximum(cp_ref[...][:, 0:1], 1.0)
    cn = jnp.maximum(cn_ref[...][:, 0:1], 1.0)
    acc = jnp.dot(sp_ref[...] / cp, wa_ref[...],
                  preferred_element_type=jnp.float32)
    acc += jnp.dot(sn_ref[...] / cn, wb_ref[...],
                   preferred_element_type=jnp.float32)
    acc += jnp.dot(z_ref[...], wc_ref[...],
                   preferred_element_type=jnp.float32)
    out_ref[...] = jnp.tanh(acc + b2_ref[...])


_MB = 1000
_GRID = (N_NODES // _MB,)


def _rows(bw):
    return pl.BlockSpec((_MB, bw), lambda i: (i, 0))


def _full(shape):
    return pl.BlockSpec(shape, lambda i: (0, 0))


def kernel(x, pos_edge_index, neg_edge_index, Wb1, bb1, Wu1, bu1,
           Wb2, bb2, Wu2, bu2):
    f32 = jnp.float32
    W1self = jnp.concatenate([Wb1[D:], Wu1[D:]], axis=1)
    Wb1t = Wb1[:D]
    Wu1t = Wu1[:D]
    z128 = jnp.zeros((D, D), f32)
    W2a = z128.at[0:H, 0:H].set(Wb2[0:H]).at[H:D, H:D].set(Wu2[0:H])
    W2b = (z128.at[0:H, H:D].set(Wu2[H:2 * H])
               .at[H:D, 0:H].set(Wb2[H:2 * H]))
    W2c = (z128.at[0:H, 0:H].set(Wb2[2 * H:3 * H])
               .at[H:D, H:D].set(Wu2[2 * H:3 * H]))
    b2 = jnp.concatenate([bb2, bu2]).reshape(1, D)
    bb1r = bb1.reshape(1, H)
    bu1r = bu1.reshape(1, H)

    padn = E_PAD - E_EDGES
    psrc = jnp.concatenate([pos_edge_index[0], jnp.zeros((padn,), jnp.int32)])
    pdst = jnp.concatenate([pos_edge_index[1],
                            jnp.full((padn,), NPAD, jnp.int32)])
    nsrc = jnp.concatenate([neg_edge_index[0], jnp.zeros((padn,), jnp.int32)])
    ndst = jnp.concatenate([neg_edge_index[1],
                            jnp.full((padn,), NPAD, jnp.int32)])

    zc128 = jnp.zeros((CW, D), f32)
    zc8 = jnp.zeros((CW, CNTW), f32)
    ones8 = jnp.ones((K, CNTW), f32)

    agg = _make_agg(False)
    spx = agg(psrc, pdst, x, zc128)
    snx = agg(nsrc, ndst, x, zc128)
    onesE = jnp.ones((E_PAD,), f32)
    cntp = jnp.zeros((NPAD, CNTW), f32).at[:, 0].set(
        jax.ops.segment_sum(onesE, pdst, num_segments=NPAD + 1)[:NPAD])
    cntn = jnp.zeros((NPAD, CNTW), f32).at[:, 0].set(
        jax.ops.segment_sum(onesE, ndst, num_segments=NPAD + 1)[:NPAD])

    ys = pl.pallas_call(
        _mm1_body,
        grid=_GRID,
        in_specs=[_rows(D), _full((D, D))],
        out_specs=_rows(D),
        out_shape=jax.ShapeDtypeStruct((N_NODES, D), f32),
    )(x, W1self)

    z = pl.pallas_call(
        _z_body,
        grid=_GRID,
        in_specs=[_rows(D), _rows(D), _rows(D), _rows(CNTW), _rows(CNTW),
                  _full((D, H)), _full((D, H)), _full((1, H)), _full((1, H))],
        out_specs=_rows(D),
        out_shape=jax.ShapeDtypeStruct((N_NODES, D), f32),
    )(spx, snx, ys, cntp, cntn, Wb1t, Wu1t, bb1r, bu1r)

    sp2 = agg(psrc, pdst, z, zc128)
    sn2 = agg(nsrc, ndst, z, zc128)

    out = pl.pallas_call(
        _f_body,
        grid=_GRID,
        in_specs=[_rows(D), _rows(D), _rows(D), _rows(CNTW), _rows(CNTW),
                  _full((D, D)), _full((D, D)), _full((D, D)),
                  _full((1, D))],
        out_specs=_rows(D),
        out_shape=jax.ShapeDtypeStruct((N_NODES, D), f32),
    )(sp2, sn2, z, cntp, cntn, W2a, W2b, W2c, b2)
    return out


# speedup vs baseline: 1.1414x; 1.0560x over previous
import functools

import jax
import jax.numpy as jnp
from jax import lax
from jax.experimental import pallas as pl
from jax.experimental.pallas import tpu as pltpu
from jax.experimental.pallas import tpu_sc as plsc

N_NODES = 50000
D = 128
H = 64
E_EDGES = 400000

NC = 2
NS = 16
NQ = 2
NPAD = 50176
E_PAD = 401408
K = 32
CW = 56
CNTW = 16
RQ = NPAD // (NC * NQ)
RT = RQ // NS
ET = E_PAD // NS
NBLK = 49
BLK = ET // NBLK
VB = BLK // 16
SELR = BLK // K + 2
KSH = K.bit_length() - 1


def _agg_body(src_hbm, dst_hbm, feat_hbm, zeros_hbm, out_hbm,
              acc, src_v, dst_v, ssrc, sdst, gbuf):
    cid = lax.axis_index("c")
    sid = lax.axis_index("s")

    tbase = sid * ET
    row0 = sid * RT
    dummy = jnp.full((16,), RQ, jnp.int32)
    zero16 = jnp.zeros((16,), jnp.int32)
    lane = lax.iota(jnp.int32, 16)

    for q in range(NQ):
        lo = (cid * NQ + q) * RQ
        hi = lo + RQ

        def zbody(i, c):
            pltpu.sync_copy(zeros_hbm, acc.at[pl.ds(row0 + i * CW, CW)])
            return c

        lax.fori_loop(0, RT // CW, zbody, jnp.int32(0))
        plsc.subcore_barrier()

        def bbody(blk, carry):
            base = tbase + blk * BLK
            pltpu.sync_copy(src_hbm.at[pl.ds(base, BLK)], src_v)
            pltpu.sync_copy(dst_hbm.at[pl.ds(base, BLK)], dst_v)

            def fbody(i, n):
                d = dst_v[pl.ds(i * 16, 16)]
                s = src_v[pl.ds(i * 16, 16)]
                m = (d >= lo) & (d < hi)
                mi = m.astype(jnp.int32)
                offs = plsc.cumsum(mi) + (n - 1)
                orow = lax.shift_right_logical(offs, KSH)
                ocol = lax.bitwise_and(offs, K - 1)
                plsc.store_scatter(ssrc, [orow, ocol], s, mask=m)
                plsc.store_scatter(sdst, [orow, ocol], d - lo, mask=m)
                return n + jnp.sum(mi)

            n_sel = lax.fori_loop(0, VB, fbody, jnp.int32(0))

            for t in range(K // 16):
                offs = lane + (n_sel + t * 16)
                orow = lax.shift_right_logical(offs, KSH)
                ocol = lax.bitwise_and(offs, K - 1)
                plsc.store_scatter(ssrc, [orow, ocol], zero16)
                plsc.store_scatter(sdst, [orow, ocol], dummy)

            nb = (n_sel + (K - 1)) // K

            def cbody(b, carry):
                pltpu.sync_copy(feat_hbm.at[ssrc.at[b]], gbuf)
                pltpu.sync_copy(gbuf, acc.at[sdst.at[b]], add=True)
                return carry

            lax.fori_loop(0, nb, cbody, jnp.int32(0))
            return carry

        lax.fori_loop(0, NBLK, bbody, jnp.int32(0))

        plsc.subcore_barrier()

        gbase = lo + sid * RT

        def wbody(i, c):
            pltpu.sync_copy(acc.at[pl.ds(row0 + i * CW, CW)],
                            out_hbm.at[pl.ds(gbase + i * CW, CW)])
            return c

        lax.fori_loop(0, RT // CW, wbody, jnp.int32(0))
        if q + 1 < NQ:
            plsc.subcore_barrier()


def _make_agg():
    mesh = plsc.VectorSubcoreMesh(core_axis_name="c", subcore_axis_name="s")
    return pl.kernel(
        _agg_body,
        out_type=jax.ShapeDtypeStruct((NPAD, D), jnp.float32),
        mesh=mesh,
        scratch_types=[
            pltpu.VMEM_SHARED((RQ + 16, D), jnp.float32),
            pltpu.VMEM((BLK,), jnp.int32),
            pltpu.VMEM((BLK,), jnp.int32),
            pltpu.VMEM((SELR, K), jnp.int32),
            pltpu.VMEM((SELR, K), jnp.int32),
            pltpu.VMEM((K, D), jnp.float32),
        ],
        compiler_params=pltpu.CompilerParams(needs_layout_passes=False),
        name="sgcn_agg",
    )


def _cnt_body(pdst_hbm, ndst_hbm, e0_hbm, e1_hbm, zeros_hbm,
              cnt_hbm, cacc, dst_v, sdst, e_v):
    cid = lax.axis_index("c")
    sid = lax.axis_index("s")

    tbase = sid * ET
    row0 = sid * RT
    dummy = jnp.full((16,), RQ, jnp.int32)
    lane = lax.iota(jnp.int32, 16)

    for q in range(NQ):
        lo = (cid * NQ + q) * RQ
        hi = lo + RQ

        def zbody(i, c):
            pltpu.sync_copy(zeros_hbm, cacc.at[pl.ds(row0 + i * CW, CW)])
            return c

        lax.fori_loop(0, RT // CW, zbody, jnp.int32(0))
        plsc.subcore_barrier()

        for dst_hbm, e_hbm in ((pdst_hbm, e0_hbm), (ndst_hbm, e1_hbm)):
            pltpu.sync_copy(e_hbm, e_v)

            def bbody(blk, carry):
                base = tbase + blk * BLK
                pltpu.sync_copy(dst_hbm.at[pl.ds(base, BLK)], dst_v)

                def fbody(i, n):
                    d = dst_v[pl.ds(i * 16, 16)]
                    m = (d >= lo) & (d < hi)
                    mi = m.astype(jnp.int32)
                    offs = plsc.cumsum(mi) + (n - 1)
                    orow = lax.shift_right_logical(offs, KSH)
                    ocol = lax.bitwise_and(offs, K - 1)
                    plsc.store_scatter(sdst, [orow, ocol], d - lo, mask=m)
                    return n + jnp.sum(mi)

                n_sel = lax.fori_loop(0, VB, fbody, jnp.int32(0))

                for t in range(K // 16):
                    offs = lane + (n_sel + t * 16)
                    orow = lax.shift_right_logical(offs, KSH)
                    ocol = lax.bitwise_and(offs, K - 1)
                    plsc.store_scatter(sdst, [orow, ocol], dummy)

                nb = (n_sel + (K - 1)) // K

                def cbody(b, carry2):
                    pltpu.sync_copy(e_v, cacc.at[sdst.at[b]], add=True)
                    return carry2

                lax.fori_loop(0, nb, cbody, jnp.int32(0))
                return carry

            lax.fori_loop(0, NBLK, bbody, jnp.int32(0))

        plsc.subcore_barrier()

        gbase = lo + sid * RT

        def wbody(i, c):
            pltpu.sync_copy(cacc.at[pl.ds(row0 + i * CW, CW)],
                            cnt_hbm.at[pl.ds(gbase + i * CW, CW)])
            return c

        lax.fori_loop(0, RT // CW, wbody, jnp.int32(0))
        if q + 1 < NQ:
            plsc.subcore_barrier()


def _make_cnt():
    mesh = plsc.VectorSubcoreMesh(core_axis_name="c", subcore_axis_name="s")
    return pl.kernel(
        _cnt_body,
        out_type=jax.ShapeDtypeStruct((NPAD, D), jnp.float32),
        mesh=mesh,
        scratch_types=[
            pltpu.VMEM_SHARED((RQ + 16, D), jnp.float32),
            pltpu.VMEM((BLK,), jnp.int32),
            pltpu.VMEM((SELR, K), jnp.int32),
            pltpu.VMEM((K, D), jnp.float32),
        ],
        compiler_params=pltpu.CompilerParams(needs_layout_passes=False),
        name="sgcn_cnt",
    )


def _mm1_body(x_ref, w_ref, ys_ref):
    ys_ref[...] = jnp.dot(x_ref[...], w_ref[...],
                          preferred_element_type=jnp.float32)


def _z_body(sp_ref, sn_ref, ys_ref, cnt_ref, wbt_ref, wut_ref,
            bb_ref, bu_ref, z_ref):
    cp = jnp.maximum(cnt_ref[...][:, 0:1], 1.0)
    cn = jnp.maximum(cnt_ref[...][:, 1:2], 1.0)
    zb = jnp.dot(sp_ref[...] / cp, wbt_ref[...],
                 preferred_element_type=jnp.float32)
    zu = jnp.dot(sn_ref[...] / cn, wut_ref[...],
                 preferred_element_type=jnp.float32)
    ys = ys_ref[...]
    zb = zb + ys[:, :H] + bb_ref[...]
    zu = zu + ys[:, H:] + bu_ref[...]
    z_ref[...] = jnp.tanh(jnp.concatenate([zb, zu], axis=1))


def _f_body(sp_ref, sn_ref, z_ref, cnt_ref, wa_ref, wb_ref, wc_ref,
            b2_ref, out_ref):
    cp = jnp.maximum(cnt_ref[...][:, 0:1], 1.0)
    cn = jnp.maximum(cnt_ref[...][:, 1:2], 1.0)
    acc = jnp.dot(sp_ref[...] / cp, wa_ref[...],
                  preferred_element_type=jnp.float32)
    acc += jnp.dot(sn_ref[...] / cn, wb_ref[...],
                   preferred_element_type=jnp.float32)
    acc += jnp.dot(z_ref[...], wc_ref[...],
                   preferred_element_type=jnp.float32)
    out_ref[...] = jnp.tanh(acc + b2_ref[...])


_MB = 1000
_GRID = (N_NODES // _MB,)


def _rows(bw):
    return pl.BlockSpec((_MB, bw), lambda i: (i, 0))


def _full(shape):
    return pl.BlockSpec(shape, lambda i: (0, 0))


def kernel(x, pos_edge_index, neg_edge_index, Wb1, bb1, Wu1, bu1,
           Wb2, bb2, Wu2, bu2):
    f32 = jnp.float32
    W1self = jnp.concatenate([Wb1[D:], Wu1[D:]], axis=1)
    Wb1t = Wb1[:D]
    Wu1t = Wu1[:D]
    z128 = jnp.zeros((D, D), f32)
    W2a = z128.at[0:H, 0:H].set(Wb2[0:H]).at[H:D, H:D].set(Wu2[0:H])
    W2b = (z128.at[0:H, H:D].set(Wu2[H:2 * H])
               .at[H:D, 0:H].set(Wb2[H:2 * H]))
    W2c = (z128.at[0:H, 0:H].set(Wb2[2 * H:3 * H])
               .at[H:D, H:D].set(Wu2[2 * H:3 * H]))
    b2 = jnp.concatenate([bb2, bu2]).reshape(1, D)
    bb1r = bb1.reshape(1, H)
    bu1r = bu1.reshape(1, H)

    padn = E_PAD - E_EDGES
    psrc = jnp.concatenate([pos_edge_index[0], jnp.zeros((padn,), jnp.int32)])
    pdst = jnp.concatenate([pos_edge_index[1],
                            jnp.full((padn,), NPAD, jnp.int32)])
    nsrc = jnp.concatenate([neg_edge_index[0], jnp.zeros((padn,), jnp.int32)])
    ndst = jnp.concatenate([neg_edge_index[1],
                            jnp.full((padn,), NPAD, jnp.int32)])

    zc128 = jnp.zeros((CW, D), f32)
    zrow = jnp.zeros((K, D), f32)
    e0 = zrow.at[:, 0].set(1.0)
    e1 = zrow.at[:, 1].set(1.0)

    cnt = _make_cnt()(pdst, ndst, e0, e1, zc128)

    agg = _make_agg()
    spx = agg(psrc, pdst, x, zc128)
    snx = agg(nsrc, ndst, x, zc128)

    ys = pl.pallas_call(
        _mm1_body,
        grid=_GRID,
        in_specs=[_rows(D), _full((D, D))],
        out_specs=_rows(D),
        out_shape=jax.ShapeDtypeStruct((N_NODES, D), f32),
    )(x, W1self)

    z = pl.pallas_call(
        _z_body,
        grid=_GRID,
        in_specs=[_rows(D), _rows(D), _rows(D), _rows(D),
                  _full((D, H)), _full((D, H)), _full((1, H)), _full((1, H))],
        out_specs=_rows(D),
        out_shape=jax.ShapeDtypeStruct((N_NODES, D), f32),
    )(spx, snx, ys, cnt, Wb1t, Wu1t, bb1r, bu1r)

    sp2 = agg(psrc, pdst, z, zc128)
    sn2 = agg(nsrc, ndst, z, zc128)

    out = pl.pallas_call(
        _f_body,
        grid=_GRID,
        in_specs=[_rows(D), _rows(D), _rows(D), _rows(D),
                  _full((D, D)), _full((D, D)), _full((D, D)),
                  _full((1, D))],
        out_specs=_rows(D),
        out_shape=jax.ShapeDtypeStruct((N_NODES, D), f32),
    )(sp2, sn2, z, cnt, W2a, W2b, W2c, b2)
    return out


# speedup vs baseline: 1.1455x; 1.0037x over previous
import functools

import jax
import jax.numpy as jnp
from jax import lax
from jax.experimental import pallas as pl
from jax.experimental.pallas import tpu as pltpu
from jax.experimental.pallas import tpu_sc as plsc

N_NODES = 50000
D = 128
H = 64
E_EDGES = 400000

NC = 2
NS = 16
NQ = 2
NPAD = 50176
E_PAD = 401408
K = 32
GC = 4
CW = 56
CNTW = 16
RQ = NPAD // (NC * NQ)
RT = RQ // NS
ET = E_PAD // NS
NBLK = 49
BLK = ET // NBLK
VB = BLK // 16
SELR = BLK // K + 2
KSH = K.bit_length() - 1


def _agg_body(edges_hbm, feat_hbm, zeros_hbm, out_hbm,
              acc, ed_v, ssrc, sdst, gbuf, gsem, asem):
    cid = lax.axis_index("c")
    sid = lax.axis_index("s")

    tbase = sid * ET
    row0 = sid * RT
    dummy = jnp.full((16,), RQ, jnp.int32)
    zero16 = jnp.zeros((16,), jnp.int32)
    lane = lax.iota(jnp.int32, 16)

    for q in range(NQ):
        lo = (cid * NQ + q) * RQ
        hi = lo + RQ

        def zbody(i, c):
            pltpu.sync_copy(zeros_hbm, acc.at[pl.ds(row0 + i * CW, CW)])
            return c

        lax.fori_loop(0, RT // CW, zbody, jnp.int32(0))
        plsc.subcore_barrier()

        def bbody(blk, carry):
            base2 = 2 * tbase + blk * (2 * BLK)
            pltpu.sync_copy(edges_hbm.at[pl.ds(base2, 2 * BLK)], ed_v)

            def fbody(i, n):
                s = ed_v[pl.ds(i * 16, 16)]
                d = ed_v[pl.ds(BLK + i * 16, 16)]
                m = (d >= lo) & (d < hi)
                mi = m.astype(jnp.int32)
                offs = plsc.cumsum(mi) + (n - 1)
                orow = lax.shift_right_logical(offs, KSH)
                ocol = lax.bitwise_and(offs, K - 1)
                plsc.store_scatter(ssrc, [orow, ocol], s, mask=m)
                plsc.store_scatter(sdst, [orow, ocol], d - lo, mask=m)
                return n + jnp.sum(mi)

            n_sel = lax.fori_loop(0, VB, fbody, jnp.int32(0))

            for t in range(K // 16):
                offs = lane + (n_sel + t * 16)
                orow = lax.shift_right_logical(offs, KSH)
                ocol = lax.bitwise_and(offs, K - 1)
                plsc.store_scatter(ssrc, [orow, ocol], zero16)
                plsc.store_scatter(sdst, [orow, ocol], dummy)

            nb = (n_sel + (K - 1)) // K

            def wave(w, carry2):
                wb = w * GC
                nw = jnp.minimum(nb - wb, GC)

                def gfire(b, c):
                    pltpu.async_copy(feat_hbm.at[ssrc.at[wb + b]],
                                     gbuf.at[pl.ds(b * K, K)], gsem)
                    return c

                lax.fori_loop(0, nw, gfire, jnp.int32(0))

                def gdrain(b, c):
                    pltpu.make_async_copy(
                        feat_hbm.at[ssrc.at[wb + b]],
                        gbuf.at[pl.ds(b * K, K)], gsem).wait()
                    return c

                lax.fori_loop(0, nw, gdrain, jnp.int32(0))

                def afire(b, c):
                    pltpu.async_copy(gbuf.at[pl.ds(b * K, K)],
                                     acc.at[sdst.at[wb + b]], asem, add=True)
                    return c

                lax.fori_loop(0, nw, afire, jnp.int32(0))

                def adrain(b, c):
                    pltpu.make_async_copy(gbuf.at[pl.ds(b * K, K)],
                                          acc.at[sdst.at[wb + b]], asem).wait()
                    return c

                lax.fori_loop(0, nw, adrain, jnp.int32(0))
                return carry2

            lax.fori_loop(0, (nb + GC - 1) // GC, wave, jnp.int32(0))
            return carry

        lax.fori_loop(0, NBLK, bbody, jnp.int32(0))

        plsc.subcore_barrier()

        gbase = lo + sid * RT

        def wbody(i, c):
            pltpu.sync_copy(acc.at[pl.ds(row0 + i * CW, CW)],
                            out_hbm.at[pl.ds(gbase + i * CW, CW)])
            return c

        lax.fori_loop(0, RT // CW, wbody, jnp.int32(0))
        if q + 1 < NQ:
            plsc.subcore_barrier()


def _make_agg():
    mesh = plsc.VectorSubcoreMesh(core_axis_name="c", subcore_axis_name="s")
    return pl.kernel(
        _agg_body,
        out_type=jax.ShapeDtypeStruct((NPAD, D), jnp.float32),
        mesh=mesh,
        scratch_types=[
            pltpu.VMEM_SHARED((RQ + 16, D), jnp.float32),
            pltpu.VMEM((2 * BLK,), jnp.int32),
            pltpu.VMEM((SELR, K), jnp.int32),
            pltpu.VMEM((SELR, K), jnp.int32),
            pltpu.VMEM((GC * K, D), jnp.float32),
            pltpu.SemaphoreType.DMA,
            pltpu.SemaphoreType.DMA,
        ],
        compiler_params=pltpu.CompilerParams(needs_layout_passes=False),
        name="sgcn_agg",
    )


def _cnt_body(pdst_hbm, ndst_hbm, e0_hbm, e1_hbm, zeros_hbm,
              cnt_hbm, cacc, dst_v, sdst, e_v):
    cid = lax.axis_index("c")
    sid = lax.axis_index("s")

    tbase = sid * ET
    row0 = sid * RT
    dummy = jnp.full((16,), RQ, jnp.int32)
    lane = lax.iota(jnp.int32, 16)

    for q in range(NQ):
        lo = (cid * NQ + q) * RQ
        hi = lo + RQ

        def zbody(i, c):
            pltpu.sync_copy(zeros_hbm, cacc.at[pl.ds(row0 + i * CW, CW)])
            return c

        lax.fori_loop(0, RT // CW, zbody, jnp.int32(0))
        plsc.subcore_barrier()

        for dst_hbm, e_hbm in ((pdst_hbm, e0_hbm), (ndst_hbm, e1_hbm)):
            pltpu.sync_copy(e_hbm, e_v)

            def bbody(blk, carry):
                base = tbase + blk * BLK
                pltpu.sync_copy(dst_hbm.at[pl.ds(base, BLK)], dst_v)

                def fbody(i, n):
                    d = dst_v[pl.ds(i * 16, 16)]
                    m = (d >= lo) & (d < hi)
                    mi = m.astype(jnp.int32)
                    offs = plsc.cumsum(mi) + (n - 1)
                    orow = lax.shift_right_logical(offs, KSH)
                    ocol = lax.bitwise_and(offs, K - 1)
                    plsc.store_scatter(sdst, [orow, ocol], d - lo, mask=m)
                    return n + jnp.sum(mi)

                n_sel = lax.fori_loop(0, VB, fbody, jnp.int32(0))

                for t in range(K // 16):
                    offs = lane + (n_sel + t * 16)
                    orow = lax.shift_right_logical(offs, KSH)
                    ocol = lax.bitwise_and(offs, K - 1)
                    plsc.store_scatter(sdst, [orow, ocol], dummy)

                nb = (n_sel + (K - 1)) // K

                def cbody(b, carry2):
                    pltpu.sync_copy(e_v, cacc.at[sdst.at[b]], add=True)
                    return carry2

                lax.fori_loop(0, nb, cbody, jnp.int32(0))
                return carry

            lax.fori_loop(0, NBLK, bbody, jnp.int32(0))

        plsc.subcore_barrier()

        gbase = lo + sid * RT

        def wbody(i, c):
            pltpu.sync_copy(cacc.at[pl.ds(row0 + i * CW, CW)],
                            cnt_hbm.at[pl.ds(gbase + i * CW, CW)])
            return c

        lax.fori_loop(0, RT // CW, wbody, jnp.int32(0))
        if q + 1 < NQ:
            plsc.subcore_barrier()


def _make_cnt():
    mesh = plsc.VectorSubcoreMesh(core_axis_name="c", subcore_axis_name="s")
    return pl.kernel(
        _cnt_body,
        out_type=jax.ShapeDtypeStruct((NPAD, D), jnp.float32),
        mesh=mesh,
        scratch_types=[
            pltpu.VMEM_SHARED((RQ + 16, D), jnp.float32),
            pltpu.VMEM((BLK,), jnp.int32),
            pltpu.VMEM((SELR, K), jnp.int32),
            pltpu.VMEM((K, D), jnp.float32),
        ],
        compiler_params=pltpu.CompilerParams(needs_layout_passes=False),
        name="sgcn_cnt",
    )


def _mm1_body(x_ref, w_ref, ys_ref):
    ys_ref[...] = jnp.dot(x_ref[...], w_ref[...],
                          preferred_element_type=jnp.float32)


def _z_body(sp_ref, sn_ref, ys_ref, cnt_ref, wbt_ref, wut_ref,
            bb_ref, bu_ref, z_ref):
    cp = jnp.maximum(cnt_ref[...][:, 0:1], 1.0)
    cn = jnp.maximum(cnt_ref[...][:, 1:2], 1.0)
    zb = jnp.dot(sp_ref[...] / cp, wbt_ref[...],
                 preferred_element_type=jnp.float32)
    zu = jnp.dot(sn_ref[...] / cn, wut_ref[...],
                 preferred_element_type=jnp.float32)
    ys = ys_ref[...]
    zb = zb + ys[:, :H] + bb_ref[...]
    zu = zu + ys[:, H:] + bu_ref[...]
    z_ref[...] = jnp.tanh(jnp.concatenate([zb, zu], axis=1))


def _f_body(sp_ref, sn_ref, z_ref, cnt_ref, wa_ref, wb_ref, wc_ref,
            b2_ref, out_ref):
    cp = jnp.maximum(cnt_ref[...][:, 0:1], 1.0)
    cn = jnp.maximum(cnt_ref[...][:, 1:2], 1.0)
    acc = jnp.dot(sp_ref[...] / cp, wa_ref[...],
                  preferred_element_type=jnp.float32)
    acc += jnp.dot(sn_ref[...] / cn, wb_ref[...],
                   preferred_element_type=jnp.float32)
    acc += jnp.dot(z_ref[...], wc_ref[...],
                   preferred_element_type=jnp.float32)
    out_ref[...] = jnp.tanh(acc + b2_ref[...])


_MB = 1000
_GRID = (N_NODES // _MB,)


def _rows(bw):
    return pl.BlockSpec((_MB, bw), lambda i: (i, 0))


def _full(shape):
    return pl.BlockSpec(shape, lambda i: (0, 0))


def kernel(x, pos_edge_index, neg_edge_index, Wb1, bb1, Wu1, bu1,
           Wb2, bb2, Wu2, bu2):
    f32 = jnp.float32
    W1self = jnp.concatenate([Wb1[D:], Wu1[D:]], axis=1)
    Wb1t = Wb1[:D]
    Wu1t = Wu1[:D]
    z128 = jnp.zeros((D, D), f32)
    W2a = z128.at[0:H, 0:H].set(Wb2[0:H]).at[H:D, H:D].set(Wu2[0:H])
    W2b = (z128.at[0:H, H:D].set(Wu2[H:2 * H])
               .at[H:D, 0:H].set(Wb2[H:2 * H]))
    W2c = (z128.at[0:H, 0:H].set(Wb2[2 * H:3 * H])
               .at[H:D, H:D].set(Wu2[2 * H:3 * H]))
    b2 = jnp.concatenate([bb2, bu2]).reshape(1, D)
    bb1r = bb1.reshape(1, H)
    bu1r = bu1.reshape(1, H)

    padn = E_PAD - E_EDGES
    psrc = jnp.concatenate([pos_edge_index[0], jnp.zeros((padn,), jnp.int32)])
    pdst = jnp.concatenate([pos_edge_index[1],
                            jnp.full((padn,), NPAD, jnp.int32)])
    nsrc = jnp.concatenate([neg_edge_index[0], jnp.zeros((padn,), jnp.int32)])
    ndst = jnp.concatenate([neg_edge_index[1],
                            jnp.full((padn,), NPAD, jnp.int32)])

    ep = jnp.stack([psrc.reshape(-1, BLK), pdst.reshape(-1, BLK)],
                   axis=1).reshape(-1)
    en = jnp.stack([nsrc.reshape(-1, BLK), ndst.reshape(-1, BLK)],
                   axis=1).reshape(-1)

    zc128 = jnp.zeros((CW, D), f32)
    zrow = jnp.zeros((K, D), f32)
    e0 = zrow.at[:, 0].set(1.0)
    e1 = zrow.at[:, 1].set(1.0)

    cnt = _make_cnt()(pdst, ndst, e0, e1, zc128)

    agg = _make_agg()
    spx = agg(ep, x, zc128)
    snx = agg(en, x, zc128)

    ys = pl.pallas_call(
        _mm1_body,
        grid=_GRID,
        in_specs=[_rows(D), _full((D, D))],
        out_specs=_rows(D),
        out_shape=jax.ShapeDtypeStruct((N_NODES, D), f32),
    )(x, W1self)

    z = pl.pallas_call(
        _z_body,
        grid=_GRID,
        in_specs=[_rows(D), _rows(D), _rows(D), _rows(D),
                  _full((D, H)), _full((D, H)), _full((1, H)), _full((1, H))],
        out_specs=_rows(D),
        out_shape=jax.ShapeDtypeStruct((N_NODES, D), f32),
    )(spx, snx, ys, cnt, Wb1t, Wu1t, bb1r, bu1r)

    sp2 = agg(ep, z, zc128)
    sn2 = agg(en, z, zc128)

    out = pl.pallas_call(
        _f_body,
        grid=_GRID,
        in_specs=[_rows(D), _rows(D), _rows(D), _rows(D),
                  _full((D, D)), _full((D, D)), _full((D, D)),
                  _full((1, D))],
        out_specs=_rows(D),
        out_shape=jax.ShapeDtypeStruct((N_NODES, D), f32),
    )(sp2, sn2, z, cnt, W2a, W2b, W2c, b2)
    return out


# speedup vs baseline: 1.1624x; 1.0147x over previous
import functools

import jax
import jax.numpy as jnp
from jax import lax
from jax.experimental import pallas as pl
from jax.experimental.pallas import tpu as pltpu
from jax.experimental.pallas import tpu_sc as plsc

N_NODES = 50000
D = 128
H = 64
E_EDGES = 400000

NC = 2
NS = 16
NQ = 2
NPAD = 50176
E_PAD = 401408
K = 32
GC = 4
ZR = 40
RQ = NPAD // (NC * NQ)
RT = RQ // NS
ET = E_PAD // NS
NBLK = 49
BLK = ET // NBLK
VB = BLK // 16
SELR = BLK // K + 2
KSH = K.bit_length() - 1


def _agg_body(edges_hbm, feat_hbm, zeros_hbm, out_hbm,
              acc, ed2, ssrc, sdst, gbuf, zbuf, gsem, asem, esem):
    cid = lax.axis_index("c")
    sid = lax.axis_index("s")

    tbase = sid * ET
    row0 = sid * RT
    dummy = jnp.full((16,), RQ, jnp.int32)
    zero16 = jnp.zeros((16,), jnp.int32)
    lane = lax.iota(jnp.int32, 16)

    pltpu.sync_copy(zeros_hbm, zbuf)

    for q in range(NQ):
        lo = (cid * NQ + q) * RQ
        hi = lo + RQ

        def zbody(i, c):
            pltpu.sync_copy(zbuf, acc.at[pl.ds(row0 + i * ZR, ZR)])
            return c

        lax.fori_loop(0, RT // ZR, zbody, jnp.int32(0))
        plsc.subcore_barrier()

        pltpu.async_copy(edges_hbm.at[pl.ds(2 * tbase, 2 * BLK)],
                         ed2.at[pl.ds(0, 2 * BLK)], esem)

        def bbody(blk, carry):
            sl = (blk % 2) * (2 * BLK)
            base2 = 2 * tbase + blk * (2 * BLK)
            nxt = jnp.minimum(blk + 1, NBLK - 1)
            nsl = ((blk + 1) % 2) * (2 * BLK)
            pltpu.async_copy(edges_hbm.at[pl.ds(2 * tbase + nxt * (2 * BLK),
                                                2 * BLK)],
                             ed2.at[pl.ds(nsl, 2 * BLK)], esem)
            pltpu.make_async_copy(edges_hbm.at[pl.ds(base2, 2 * BLK)],
                                  ed2.at[pl.ds(sl, 2 * BLK)], esem).wait()

            def fbody(i, n):
                s = ed2[pl.ds(sl + i * 16, 16)]
                d = ed2[pl.ds(sl + BLK + i * 16, 16)]
                m = (d >= lo) & (d < hi)
                mi = m.astype(jnp.int32)
                offs = plsc.cumsum(mi) + (n - 1)
                orow = lax.shift_right_logical(offs, KSH)
                ocol = lax.bitwise_and(offs, K - 1)
                plsc.store_scatter(ssrc, [orow, ocol], s, mask=m)
                plsc.store_scatter(sdst, [orow, ocol], d - lo, mask=m)
                return n + jnp.sum(mi)

            n_sel = lax.fori_loop(0, VB, fbody, jnp.int32(0))

            for t in range(K // 16):
                offs = lane + (n_sel + t * 16)
                orow = lax.shift_right_logical(offs, KSH)
                ocol = lax.bitwise_and(offs, K - 1)
                plsc.store_scatter(ssrc, [orow, ocol], zero16)
                plsc.store_scatter(sdst, [orow, ocol], dummy)

            nb = (n_sel + (K - 1)) // K

            def wave(w, carry2):
                wb = w * GC
                nw = jnp.minimum(nb - wb, GC)

                def gfire(b, c):
                    pltpu.async_copy(feat_hbm.at[ssrc.at[wb + b]],
                                     gbuf.at[pl.ds(b * K, K)], gsem)
                    return c

                lax.fori_loop(0, nw, gfire, jnp.int32(0))

                def gdrain(b, c):
                    pltpu.make_async_copy(
                        feat_hbm.at[ssrc.at[wb + b]],
                        gbuf.at[pl.ds(b * K, K)], gsem).wait()
                    return c

                lax.fori_loop(0, nw, gdrain, jnp.int32(0))

                def afire(b, c):
                    pltpu.async_copy(gbuf.at[pl.ds(b * K, K)],
                                     acc.at[sdst.at[wb + b]], asem, add=True)
                    return c

                lax.fori_loop(0, nw, afire, jnp.int32(0))

                def adrain(b, c):
                    pltpu.make_async_copy(gbuf.at[pl.ds(b * K, K)],
                                          acc.at[sdst.at[wb + b]], asem).wait()
                    return c

                lax.fori_loop(0, nw, adrain, jnp.int32(0))
                return carry2

            lax.fori_loop(0, (nb + GC - 1) // GC, wave, jnp.int32(0))
            return carry

        lax.fori_loop(0, NBLK, bbody, jnp.int32(0))

        pltpu.make_async_copy(
            edges_hbm.at[pl.ds(2 * tbase + (NBLK - 1) * (2 * BLK), 2 * BLK)],
            ed2.at[pl.ds((NBLK % 2) * (2 * BLK), 2 * BLK)], esem).wait()

        plsc.subcore_barrier()

        pltpu.sync_copy(acc.at[pl.ds(row0, RT)],
                        out_hbm.at[pl.ds(lo + sid * RT, RT)])
        if q + 1 < NQ:
            plsc.subcore_barrier()


def _make_agg():
    mesh = plsc.VectorSubcoreMesh(core_axis_name="c", subcore_axis_name="s")
    return pl.kernel(
        _agg_body,
        out_type=jax.ShapeDtypeStruct((NPAD, D), jnp.float32),
        mesh=mesh,
        scratch_types=[
            pltpu.VMEM_SHARED((RQ + 16, D), jnp.float32),
            pltpu.VMEM((4 * BLK,), jnp.int32),
            pltpu.VMEM((SELR, K), jnp.int32),
            pltpu.VMEM((SELR, K), jnp.int32),
            pltpu.VMEM((GC * K, D), jnp.float32),
            pltpu.VMEM((ZR, D), jnp.float32),
            pltpu.SemaphoreType.DMA,
            pltpu.SemaphoreType.DMA,
            pltpu.SemaphoreType.DMA,
        ],
        compiler_params=pltpu.CompilerParams(needs_layout_passes=False),
        name="sgcn_agg",
    )


def _cnt_body(pdst_hbm, ndst_hbm, e0_hbm, e1_hbm, zeros_hbm,
              cnt_hbm, cacc, dst_v, sdst, e_v):
    cid = lax.axis_index("c")
    sid = lax.axis_index("s")

    tbase = sid * ET
    row0 = sid * RT
    dummy = jnp.full((16,), RQ, jnp.int32)
    lane = lax.iota(jnp.int32, 16)

    for q in range(NQ):
        lo = (cid * NQ + q) * RQ
        hi = lo + RQ

        def zbody(i, c):
            pltpu.sync_copy(zeros_hbm, cacc.at[pl.ds(row0 + i * ZR, ZR)])
            return c

        lax.fori_loop(0, RT // ZR, zbody, jnp.int32(0))
        plsc.subcore_barrier()

        for dst_hbm, e_hbm in ((pdst_hbm, e0_hbm), (ndst_hbm, e1_hbm)):
            pltpu.sync_copy(e_hbm, e_v)

            def bbody(blk, carry):
                base = tbase + blk * BLK
                pltpu.sync_copy(dst_hbm.at[pl.ds(base, BLK)], dst_v)

                def fbody(i, n):
                    d = dst_v[pl.ds(i * 16, 16)]
                    m = (d >= lo) & (d < hi)
                    mi = m.astype(jnp.int32)
                    offs = plsc.cumsum(mi) + (n - 1)
                    orow = lax.shift_right_logical(offs, KSH)
                    ocol = lax.bitwise_and(offs, K - 1)
                    plsc.store_scatter(sdst, [orow, ocol], d - lo, mask=m)
                    return n + jnp.sum(mi)

                n_sel = lax.fori_loop(0, VB, fbody, jnp.int32(0))

                for t in range(K // 16):
                    offs = lane + (n_sel + t * 16)
                    orow = lax.shift_right_logical(offs, KSH)
                    ocol = lax.bitwise_and(offs, K - 1)
                    plsc.store_scatter(sdst, [orow, ocol], dummy)

                nb = (n_sel + (K - 1)) // K

                def cbody(b, carry2):
                    pltpu.sync_copy(e_v, cacc.at[sdst.at[b]], add=True)
                    return carry2

                lax.fori_loop(0, nb, cbody, jnp.int32(0))
                return carry

            lax.fori_loop(0, NBLK, bbody, jnp.int32(0))

        plsc.subcore_barrier()

        pltpu.sync_copy(cacc.at[pl.ds(row0, RT)],
                        cnt_hbm.at[pl.ds(lo + sid * RT, RT)])
        if q + 1 < NQ:
            plsc.subcore_barrier()


def _make_cnt():
    mesh = plsc.VectorSubcoreMesh(core_axis_name="c", subcore_axis_name="s")
    return pl.kernel(
        _cnt_body,
        out_type=jax.ShapeDtypeStruct((NPAD, D), jnp.float32),
        mesh=mesh,
        scratch_types=[
            pltpu.VMEM_SHARED((RQ + 16, D), jnp.float32),
            pltpu.VMEM((BLK,), jnp.int32),
            pltpu.VMEM((SELR, K), jnp.int32),
            pltpu.VMEM((K, D), jnp.float32),
        ],
        compiler_params=pltpu.CompilerParams(needs_layout_passes=False),
        name="sgcn_cnt",
    )


def _mm1_body(x_ref, w_ref, ys_ref):
    ys_ref[...] = jnp.dot(x_ref[...], w_ref[...],
                          preferred_element_type=jnp.float32)


def _z_body(sp_ref, sn_ref, ys_ref, cnt_ref, wbt_ref, wut_ref,
            bb_ref, bu_ref, z_ref):
    cp = jnp.maximum(cnt_ref[...][:, 0:1], 1.0)
    cn = jnp.maximum(cnt_ref[...][:, 1:2], 1.0)
    zb = jnp.dot(sp_ref[...] / cp, wbt_ref[...],
                 preferred_element_type=jnp.float32)
    zu = jnp.dot(sn_ref[...] / cn, wut_ref[...],
                 preferred_element_type=jnp.float32)
    ys = ys_ref[...]
    zb = zb + ys[:, :H] + bb_ref[...]
    zu = zu + ys[:, H:] + bu_ref[...]
    z_ref[...] = jnp.tanh(jnp.concatenate([zb, zu], axis=1))


def _f_body(sp_ref, sn_ref, z_ref, cnt_ref, wa_ref, wb_ref, wc_ref,
            b2_ref, out_ref):
    cp = jnp.maximum(cnt_ref[...][:, 0:1], 1.0)
    cn = jnp.maximum(cnt_ref[...][:, 1:2], 1.0)
    acc = jnp.dot(sp_ref[...] / cp, wa_ref[...],
                  preferred_element_type=jnp.float32)
    acc += jnp.dot(sn_ref[...] / cn, wb_ref[...],
                   preferred_element_type=jnp.float32)
    acc += jnp.dot(z_ref[...], wc_ref[...],
                   preferred_element_type=jnp.float32)
    out_ref[...] = jnp.tanh(acc + b2_ref[...])


_MB = 1000
_GRID = (N_NODES // _MB,)


def _rows(bw):
    return pl.BlockSpec((_MB, bw), lambda i: (i, 0))


def _full(shape):
    return pl.BlockSpec(shape, lambda i: (0, 0))


def kernel(x, pos_edge_index, neg_edge_index, Wb1, bb1, Wu1, bu1,
           Wb2, bb2, Wu2, bu2):
    f32 = jnp.float32
    W1self = jnp.concatenate([Wb1[D:], Wu1[D:]], axis=1)
    Wb1t = Wb1[:D]
    Wu1t = Wu1[:D]
    z128 = jnp.zeros((D, D), f32)
    W2a = z128.at[0:H, 0:H].set(Wb2[0:H]).at[H:D, H:D].set(Wu2[0:H])
    W2b = (z128.at[0:H, H:D].set(Wu2[H:2 * H])
               .at[H:D, 0:H].set(Wb2[H:2 * H]))
    W2c = (z128.at[0:H, 0:H].set(Wb2[2 * H:3 * H])
               .at[H:D, H:D].set(Wu2[2 * H:3 * H]))
    b2 = jnp.concatenate([bb2, bu2]).reshape(1, D)
    bb1r = bb1.reshape(1, H)
    bu1r = bu1.reshape(1, H)

    padn = E_PAD - E_EDGES
    psrc = jnp.concatenate([pos_edge_index[0], jnp.zeros((padn,), jnp.int32)])
    pdst = jnp.concatenate([pos_edge_index[1],
                            jnp.full((padn,), NPAD, jnp.int32)])
    nsrc = jnp.concatenate([neg_edge_index[0], jnp.zeros((padn,), jnp.int32)])
    ndst = jnp.concatenate([neg_edge_index[1],
                            jnp.full((padn,), NPAD, jnp.int32)])

    ep = jnp.stack([psrc.reshape(-1, BLK), pdst.reshape(-1, BLK)],
                   axis=1).reshape(-1)
    en = jnp.stack([nsrc.reshape(-1, BLK), ndst.reshape(-1, BLK)],
                   axis=1).reshape(-1)

    zc128 = jnp.zeros((ZR, D), f32)
    zrow = jnp.zeros((K, D), f32)
    e0 = zrow.at[:, 0].set(1.0)
    e1 = zrow.at[:, 1].set(1.0)

    cnt = _make_cnt()(pdst, ndst, e0, e1, zc128)

    agg = _make_agg()
    spx = agg(ep, x, zc128)
    snx = agg(en, x, zc128)

    ys = pl.pallas_call(
        _mm1_body,
        grid=_GRID,
        in_specs=[_rows(D), _full((D, D))],
        out_specs=_rows(D),
        out_shape=jax.ShapeDtypeStruct((N_NODES, D), f32),
    )(x, W1self)

    z = pl.pallas_call(
        _z_body,
        grid=_GRID,
        in_specs=[_rows(D), _rows(D), _rows(D), _rows(D),
                  _full((D, H)), _full((D, H)), _full((1, H)), _full((1, H))],
        out_specs=_rows(D),
        out_shape=jax.ShapeDtypeStruct((N_NODES, D), f32),
    )(spx, snx, ys, cnt, Wb1t, Wu1t, bb1r, bu1r)

    sp2 = agg(ep, z, zc128)
    sn2 = agg(en, z, zc128)

    out = pl.pallas_call(
        _f_body,
        grid=_GRID,
        in_specs=[_rows(D), _rows(D), _rows(D), _rows(D),
                  _full((D, D)), _full((D, D)), _full((D, D)),
                  _full((1, D))],
        out_specs=_rows(D),
        out_shape=jax.ShapeDtypeStruct((N_NODES, D), f32),
    )(sp2, sn2, z, cnt, W2a, W2b, W2c, b2)
    return out
